# baseline (speedup 1.0000x reference)
import functools

import jax
import jax.numpy as jnp
from jax import lax
from jax.experimental import pallas as pl
from jax.experimental.pallas import tpu as pltpu
from jax.experimental.pallas import tpu_sc as plsc

N = 10000
E = 320000
D = 128

NC = 2
NS = 16
NW = NC * NS
L = 16

NCH = E // 128
CB = 20
IBLK = 4
CPW = IBLK * CB
EB = CB * 128
NPAD = 10112
RPT = NPAD // NS

_mesh = plsc.VectorSubcoreMesh(core_axis_name="c", subcore_axis_name="s")
_sc_params = pltpu.CompilerParams(needs_layout_passes=False)


@functools.partial(
    pl.kernel,
    mesh=_mesh,
    out_type=jax.ShapeDtypeStruct((NW, NPAD), jnp.float32),
    compiler_params=_sc_params,
    scratch_types=[
        pltpu.VMEM((EB,), jnp.int32),
        pltpu.VMEM((NPAD,), jnp.float32),
    ],
)
def _deg_kernel(ef_hbm, out_hbm, d_v, hist_v):
    cid = lax.axis_index("c")
    sid = lax.axis_index("s")
    wid = sid * NC + cid
    zeros16 = jnp.zeros((L,), jnp.float32)
    ones16 = jnp.ones((L,), jnp.float32)

    def zbody(i, c):
        hist_v[pl.ds(i * L, L)] = zeros16
        return c

    lax.fori_loop(0, NPAD // L, zbody, 0)
    ebase = E + wid * CPW * 128
    nblk = jnp.where(wid == NW - 1, 1, IBLK)

    def block(bk, c):
        pltpu.sync_copy(ef_hbm.at[pl.ds(ebase + bk * EB, EB)], d_v)

        def body(i, c2):
            d = d_v[pl.ds(i * L, L)]
            plsc.addupdate_scatter(hist_v, [d], ones16)
            return c2

        lax.fori_loop(0, EB // L, body, 0)
        return c

    lax.fori_loop(0, nblk, block, 0)
    pltpu.sync_copy(hist_v, out_hbm.at[wid])


@functools.partial(
    pl.kernel,
    mesh=_mesh,
    out_type=jax.ShapeDtypeStruct((NC, NPAD, D), jnp.float32),
    compiler_params=_sc_params,
    scratch_types=[
        pltpu.VMEM((EB,), jnp.int32),
        pltpu.VMEM((EB,), jnp.int32),
        pltpu.VMEM((2, 128, D), jnp.float32),
        pltpu.VMEM_SHARED((NPAD, D), jnp.float32),
        pltpu.SemaphoreType.DMA,
        pltpu.SemaphoreType.DMA,
    ],
)
def _scatter_kernel(p_hbm, ef_hbm, out_hbm,
                    si_v, di_v, rows_v, s_sh, sem_a, sem_b):
    cid = lax.axis_index("c")
    sid = lax.axis_index("s")
    wid = sid * NC + cid

    zeros16 = jnp.zeros((L,), jnp.float32)
    zbuf = rows_v.at[0]

    def zb(i, c):
        zbuf[i // 8, pl.ds((i % 8) * L, L)] = zeros16
        return c

    lax.fori_loop(0, 128 * (D // L), zb, 0)
    r0 = sid * RPT
    for k in range(RPT // 128):
        pltpu.sync_copy(zbuf, s_sh.at[pl.ds(r0 + k * 128, 128)])
    rem = RPT % 128
    if rem:
        pltpu.sync_copy(zbuf.at[pl.ds(0, rem)],
                        s_sh.at[pl.ds(r0 + (RPT // 128) * 128, rem)])
    plsc.subcore_barrier()

    buf_a, buf_b = rows_v.at[0], rows_v.at[1]

    def start(j, buf, sem):
        pltpu.async_copy(p_hbm.at[si_v.at[pl.ds(j * 128, 128)]], buf, sem)

    def drain(buf, sem):
        pltpu.make_async_copy(p_hbm.at[pl.ds(0, 128)], buf, sem).wait()

    def scat(j, buf):
        pltpu.sync_copy(buf, s_sh.at[di_v.at[pl.ds(j * 128, 128)]], add=True)

    ebase = wid * CPW * 128
    nblk = jnp.where(wid == NW - 1, 1, IBLK)

    def block(b, c):
        pltpu.sync_copy(ef_hbm.at[pl.ds(ebase + b * EB, EB)], si_v)
        pltpu.sync_copy(ef_hbm.at[pl.ds(E + ebase + b * EB, EB)], di_v)
        start(0, buf_a, sem_a)

        def body(t, c2):
            start(2 * t + 1, buf_b, sem_b)
            drain(buf_a, sem_a)
            scat(2 * t, buf_a)
            start(2 * t + 2, buf_a, sem_a)
            drain(buf_b, sem_b)
            scat(2 * t + 1, buf_b)
            return c2

        lax.fori_loop(0, CB // 2 - 1, body, 0)
        start(CB - 1, buf_b, sem_b)
        drain(buf_a, sem_a)
        scat(CB - 2, buf_a)
        drain(buf_b, sem_b)
        scat(CB - 1, buf_b)
        return c

    lax.fori_loop(0, nblk, block, 0)
    plsc.subcore_barrier()
    pltpu.sync_copy(s_sh.at[pl.ds(r0, RPT)], out_hbm.at[cid, pl.ds(r0, RPT)])


def _dinv_rows(parts):
    ones = jnp.ones((NW, 128), jnp.float32)
    deg = lax.dot_general(parts, ones, (((0,), (0,)), ((), ())),
                          preferred_element_type=jnp.float32)
    return lax.rsqrt(deg + 1.0)[:N]


def _h_body(x_ref, w_ref, h_ref):
    h_ref[...] = jnp.dot(x_ref[...], w_ref[...],
                         preferred_element_type=jnp.float32)


_h_kernel = pl.pallas_call(
    _h_body,
    out_shape=jax.ShapeDtypeStruct((N, D), jnp.float32),
)


def _mm_body(h_ref, parts_ref, p_ref):
    p_ref[...] = h_ref[...] * _dinv_rows(parts_ref[...])


_mm_kernel = pl.pallas_call(
    _mm_body,
    out_shape=jax.ShapeDtypeStruct((N, D), jnp.float32),
)


def _fin_body(s_ref, p_ref, parts_ref, x_ref, b_ref, o_ref):
    dinv = _dinv_rows(parts_ref[...])
    s = (s_ref[0] + s_ref[1])[:N]
    agg = dinv * (s + p_ref[...]) + b_ref[...]
    o_ref[...] = jnp.maximum(agg, 0.0) + x_ref[...]


_fin_kernel = pl.pallas_call(
    _fin_body,
    out_shape=jax.ShapeDtypeStruct((N, D), jnp.float32),
)


def kernel(x, edge_index, W, b):
    ef = edge_index.reshape(2 * E)
    h = _h_kernel(x, W)
    parts = _deg_kernel(ef)
    p = _mm_kernel(h, parts)
    s = _scatter_kernel(p, ef)
    return _fin_kernel(s, p, parts, x, b.reshape(1, D))


# speedup vs baseline: 1.0050x; 1.0050x over previous
import functools

import jax
import jax.numpy as jnp
from jax import lax
from jax.experimental import pallas as pl
from jax.experimental.pallas import tpu as pltpu
from jax.experimental.pallas import tpu_sc as plsc

N = 10000
E = 320000
D = 128

NC = 2
NS = 16
NW = NC * NS
L = 16

NCH = E // 128
CB = 20
IBLK = 4
CPW = IBLK * CB
EB = CB * 128
NPAD = 10112
RPT = NPAD // NS

_mesh = plsc.VectorSubcoreMesh(core_axis_name="c", subcore_axis_name="s")
_sc_params = pltpu.CompilerParams(needs_layout_passes=False)


@functools.partial(
    pl.kernel,
    mesh=_mesh,
    out_type=jax.ShapeDtypeStruct((NW, NPAD), jnp.float32),
    compiler_params=_sc_params,
    scratch_types=[
        pltpu.VMEM((EB,), jnp.int32),
        pltpu.VMEM((NPAD,), jnp.float32),
    ],
)
def _deg_kernel(ef_hbm, out_hbm, d_v, hist_v):
    cid = lax.axis_index("c")
    sid = lax.axis_index("s")
    wid = sid * NC + cid
    zeros16 = jnp.zeros((L,), jnp.float32)
    ones16 = jnp.ones((L,), jnp.float32)

    def zbody(i, c):
        hist_v[pl.ds(i * L, L)] = zeros16
        return c

    lax.fori_loop(0, NPAD // L, zbody, 0)
    ebase = E + wid * CPW * 128
    nblk = jnp.where(wid == NW - 1, 1, IBLK)

    def block(bk, c):
        pltpu.sync_copy(ef_hbm.at[pl.ds(ebase + bk * EB, EB)], d_v)

        def body(i, c2):
            d = d_v[pl.ds(i * L, L)]
            plsc.addupdate_scatter(hist_v, [d], ones16)
            return c2

        lax.fori_loop(0, EB // L, body, 0)
        return c

    lax.fori_loop(0, nblk, block, 0)
    pltpu.sync_copy(hist_v, out_hbm.at[wid])


@functools.partial(
    pl.kernel,
    mesh=_mesh,
    out_type=jax.ShapeDtypeStruct((NC, NPAD, D), jnp.float32),
    compiler_params=_sc_params,
    scratch_types=[
        pltpu.VMEM((EB,), jnp.int32),
        pltpu.VMEM((EB,), jnp.int32),
        pltpu.VMEM((2, 128, D), jnp.float32),
        pltpu.VMEM_SHARED((NPAD, D), jnp.float32),
        pltpu.SemaphoreType.DMA,
        pltpu.SemaphoreType.DMA,
    ],
)
def _scatter_kernel(p_hbm, ef_hbm, out_hbm,
                    si_v, di_v, rows_v, s_sh, sem_a, sem_b):
    cid = lax.axis_index("c")
    sid = lax.axis_index("s")
    wid = sid * NC + cid

    zeros16 = jnp.zeros((L,), jnp.float32)
    zbuf = rows_v.at[0]

    def zb(i, c):
        zbuf[i // 8, pl.ds((i % 8) * L, L)] = zeros16
        return c

    lax.fori_loop(0, 128 * (D // L), zb, 0)
    r0 = sid * RPT
    for k in range(RPT // 128):
        pltpu.sync_copy(zbuf, s_sh.at[pl.ds(r0 + k * 128, 128)])
    rem = RPT % 128
    if rem:
        pltpu.sync_copy(zbuf.at[pl.ds(0, rem)],
                        s_sh.at[pl.ds(r0 + (RPT // 128) * 128, rem)])
    plsc.subcore_barrier()

    buf_a, buf_b = rows_v.at[0], rows_v.at[1]

    def start(j, buf, sem):
        pltpu.async_copy(p_hbm.at[si_v.at[pl.ds(j * 128, 128)]], buf, sem)

    def drain(buf, sem):
        pltpu.make_async_copy(p_hbm.at[pl.ds(0, 128)], buf, sem).wait()

    def scat(j, buf):
        pltpu.sync_copy(buf, s_sh.at[di_v.at[pl.ds(j * 128, 128)]], add=True)

    ebase = wid * CPW * 128
    nblk = jnp.where(wid == NW - 1, 1, IBLK)

    def block(b, c):
        pltpu.sync_copy(ef_hbm.at[pl.ds(ebase + b * EB, EB)], si_v)
        pltpu.sync_copy(ef_hbm.at[pl.ds(E + ebase + b * EB, EB)], di_v)
        start(0, buf_a, sem_a)

        def body(t, c2):
            start(2 * t + 1, buf_b, sem_b)
            drain(buf_a, sem_a)
            scat(2 * t, buf_a)
            start(2 * t + 2, buf_a, sem_a)
            drain(buf_b, sem_b)
            scat(2 * t + 1, buf_b)
            return c2

        lax.fori_loop(0, CB // 2 - 1, body, 0)
        start(CB - 1, buf_b, sem_b)
        drain(buf_a, sem_a)
        scat(CB - 2, buf_a)
        drain(buf_b, sem_b)
        scat(CB - 1, buf_b)
        return c

    lax.fori_loop(0, nblk, block, 0)
    plsc.subcore_barrier()
    pltpu.sync_copy(s_sh.at[pl.ds(r0, RPT)], out_hbm.at[cid, pl.ds(r0, RPT)])


def _dinv_rows(parts):
    ones = jnp.ones((NW, 128), jnp.float32)
    deg = lax.dot_general(parts, ones, (((0,), (0,)), ((), ())),
                          preferred_element_type=jnp.float32)
    return lax.rsqrt(deg + 1.0)[:N]


def _mm_body(x_ref, w_ref, parts_ref, p_ref):
    dinv = _dinv_rows(parts_ref[...])
    h = jnp.dot(x_ref[...], w_ref[...], preferred_element_type=jnp.float32)
    p_ref[...] = h * dinv


_mm_kernel = pl.pallas_call(
    _mm_body,
    out_shape=jax.ShapeDtypeStruct((N, D), jnp.float32),
)


def _fin_body(s_ref, p_ref, parts_ref, x_ref, b_ref, o_ref):
    dinv = _dinv_rows(parts_ref[...])
    s = (s_ref[0] + s_ref[1])[:N]
    agg = dinv * (s + p_ref[...]) + b_ref[...]
    o_ref[...] = jnp.maximum(agg, 0.0) + x_ref[...]


_fin_kernel = pl.pallas_call(
    _fin_body,
    out_shape=jax.ShapeDtypeStruct((N, D), jnp.float32),
)


def kernel(x, edge_index, W, b):
    ef = edge_index.reshape(2 * E)
    parts = _deg_kernel(ef)
    p = _mm_kernel(x, W, parts)
    s = _scatter_kernel(p, ef)
    return _fin_kernel(s, p, parts, x, b.reshape(1, D))
